# Initial kernel scaffold; baseline (speedup 1.0000x reference)
import math

import jax
import jax.numpy as jnp
from jax import lax
from jax.experimental import pallas as pl
from jax.experimental.pallas import tpu as pltpu
from jax.experimental.pallas import tpu_sc as plsc

_N = 32768
_LANES = 16
_NCH = _N // _LANES
_BETA = 2.0 / 3.0
_MAGIC = 0.8
_X0 = (0.0 - (-0.1)) / (1.1 - (-0.1))
_LOGITS_BETA = (math.log(_X0) - math.log(1.0 - _X0)) * _BETA
_EPS = 1e-06
_INT_MIN = jnp.int32(-2147483648)
_M31 = jnp.int32(0x7FFFFFFF)


def _body(la_hbm, out_hbm, la_v, z_v, key_v, hist_v):
    cid = lax.axis_index("c")
    sid = lax.axis_index("s")

    @pl.when(jnp.logical_and(cid == 0, sid == 0))
    def _():
        pltpu.sync_copy(la_hbm, la_v)
        lane = lax.iota(jnp.int32, 16)

        def p1(i, acc):
            x = la_v[pl.ds(i * 16, 16)]
            at = jnp.clip(x - jnp.float32(_LOGITS_BETA), -15.0, 15.0)
            lv = jnp.clip(1.0 / (1.0 + jnp.exp(-at)), _EPS, 1.0 - _EPS)
            u = x / jnp.float32(_BETA) * jnp.float32(_MAGIC)
            z_v[pl.ds(i * 16, 16)] = 1.0 / (1.0 + jnp.exp(-u))
            b = lax.bitcast_convert_type(x, jnp.int32)
            s = lax.shift_right_logical(b, 31)
            key_v[pl.ds(i * 16, 16)] = b ^ (s * _M31)
            return acc + lv

        accv = lax.fori_loop(0, _NCH, p1, jnp.zeros((16,), jnp.float32))
        lc = jnp.sum(accv)

        t_i = lc.astype(jnp.int32)
        frac = lc - t_i.astype(jnp.float32)
        add1 = jnp.logical_or(
            frac > 0.5, jnp.logical_and(frac == 0.5, (t_i & 1) == 1)
        ).astype(jnp.int32)
        k = jnp.maximum(jnp.int32(1), t_i + add1)
        num_zeros = jnp.int32(_N) - k

        r_res = num_zeros
        pref = jnp.int32(0)
        for rnd in range(4):
            sh = 24 - 8 * rnd

            def zero_hist(j, _):
                hist_v[pl.ds(j * 16, 16)] = jnp.zeros((16,), jnp.int32)
                return 0

            lax.fori_loop(0, 256, zero_hist, 0)

            ones = jnp.ones((16,), jnp.int32)

            if rnd == 0:

                def scan0(i, _):
                    key = key_v[pl.ds(i * 16, 16)]
                    ux = key ^ _INT_MIN
                    byte = lax.shift_right_logical(ux, sh) & jnp.int32(255)
                    plsc.addupdate_scatter(hist_v, [lane * 256 + byte], ones)
                    return 0

                lax.fori_loop(0, _NCH, scan0, 0)
            else:

                def scanr(i, _, sh=sh, pref=pref):
                    key = key_v[pl.ds(i * 16, 16)]
                    ux = key ^ _INT_MIN
                    match = lax.shift_right_logical(ux, sh + 8) == pref
                    byte = lax.shift_right_logical(ux, sh) & jnp.int32(255)
                    plsc.addupdate_scatter(
                        hist_v, [lane * 256 + byte], ones, mask=match
                    )
                    return 0

                lax.fori_loop(0, _NCH, scanr, 0)

            def merge(cb, carry):
                bcount, cum_before, total = carry

                def lsum(l, acc):
                    return acc + hist_v[pl.ds(l * 256 + cb * 16, 16)]

                mchunk = lax.fori_loop(0, 16, lsum, jnp.zeros((16,), jnp.int32))
                cum = plsc.cumsum(mchunk) + total
                sel = cum <= r_res
                bcount = bcount + jnp.sum(sel.astype(jnp.int32))
                cum_before = jnp.maximum(
                    cum_before, jnp.max(jnp.where(sel, cum, jnp.int32(0)))
                )
                total = total + jnp.sum(mchunk)
                return bcount, cum_before, total

            bidx, cum_before, _tot = lax.fori_loop(
                0, 16, merge, (jnp.int32(0), jnp.int32(0), jnp.int32(0))
            )
            pref = (pref * jnp.int32(256)) | bidx
            r_res = r_res - cum_before

        t_key = pref ^ _INT_MIN
        need = r_res

        def zpass(i, carry):
            key = key_v[pl.ds(i * 16, 16)]
            z = z_v[pl.ds(i * 16, 16)]
            ltm = key < t_key
            eqm = key == t_key
            m = eqm.astype(jnp.int32)
            c = plsc.cumsum(m)
            ord_excl = carry + (c - m)
            zero = jnp.logical_or(ltm, jnp.logical_and(eqm, ord_excl < need))
            z_v[pl.ds(i * 16, 16)] = jnp.where(zero, jnp.float32(0.0), z)
            return carry + jnp.sum(m)

        lax.fori_loop(0, _NCH, zpass, jnp.int32(0))

        pltpu.sync_copy(z_v, out_hbm)


_mask_kernel = pl.kernel(
    _body,
    out_type=jax.ShapeDtypeStruct((_N,), jnp.float32),
    mesh=plsc.VectorSubcoreMesh(core_axis_name="c", subcore_axis_name="s"),
    scratch_types=[
        pltpu.VMEM((_N,), jnp.float32),
        pltpu.VMEM((_N,), jnp.float32),
        pltpu.VMEM((_N,), jnp.int32),
        pltpu.VMEM((4096,), jnp.int32),
    ],
)


def kernel(log_alpha):
    return _mask_kernel(log_alpha)


# speedup vs baseline: 1.0890x; 1.0890x over previous
import math

import jax
import jax.numpy as jnp
import numpy as np
from jax import lax
from jax.experimental import pallas as pl
from jax.experimental.pallas import tpu as pltpu
from jax.experimental.pallas import tpu_sc as plsc

_N = 32768
_LANES = 16
_NCH = _N // _LANES
_BETA = 2.0 / 3.0
_MAGIC = 0.8
_X0 = (0.0 - (-0.1)) / (1.1 - (-0.1))
_LOGITS_BETA = (math.log(_X0) - math.log(1.0 - _X0)) * _BETA
_EPS = 1e-06
_INT_MIN = np.int32(-2147483648)
_M31 = np.int32(0x7FFFFFFF)


def _body(la_hbm, out_hbm, la_v, z_v, key_v, hist_v):
    cid = lax.axis_index("c")
    sid = lax.axis_index("s")

    @pl.when(jnp.logical_and(cid == 0, sid == 0))
    def _():
        pltpu.sync_copy(la_hbm, la_v)
        lane = lax.iota(jnp.int32, 16)

        def p1(i, acc):
            x = la_v[pl.ds(i * 16, 16)]
            at = jnp.clip(x - jnp.float32(_LOGITS_BETA), -15.0, 15.0)
            lv = jnp.clip(1.0 / (1.0 + jnp.exp(-at)), _EPS, 1.0 - _EPS)
            u = x / jnp.float32(_BETA) * jnp.float32(_MAGIC)
            z_v[pl.ds(i * 16, 16)] = 1.0 / (1.0 + jnp.exp(-u))
            b = lax.bitcast_convert_type(x, jnp.int32)
            s = lax.shift_right_logical(b, 31)
            key_v[pl.ds(i * 16, 16)] = b ^ (s * _M31)
            return acc + lv

        accv = lax.fori_loop(0, _NCH, p1, jnp.zeros((16,), jnp.float32))
        lc = jnp.sum(accv)

        t_i = lc.astype(jnp.int32)
        frac = lc - t_i.astype(jnp.float32)
        add1 = jnp.logical_or(
            frac > 0.5, jnp.logical_and(frac == 0.5, (t_i & 1) == 1)
        ).astype(jnp.int32)
        k = jnp.maximum(jnp.int32(1), t_i + add1)
        num_zeros = jnp.int32(_N) - k

        r_res = num_zeros
        pref = jnp.int32(0)
        for rnd in range(4):
            sh = 24 - 8 * rnd

            def zero_hist(j, _):
                hist_v[pl.ds(j * 16, 16)] = jnp.zeros((16,), jnp.int32)
                return 0

            lax.fori_loop(0, 256, zero_hist, 0)

            ones = jnp.ones((16,), jnp.int32)

            if rnd == 0:

                def scan0(i, _):
                    key = key_v[pl.ds(i * 16, 16)]
                    ux = key ^ _INT_MIN
                    byte = lax.shift_right_logical(ux, sh) & jnp.int32(255)
                    plsc.addupdate_scatter(hist_v, [lane * 256 + byte], ones)
                    return 0

                lax.fori_loop(0, _NCH, scan0, 0)
            else:

                def scanr(i, _, sh=sh, pref=pref):
                    key = key_v[pl.ds(i * 16, 16)]
                    ux = key ^ _INT_MIN
                    match = lax.shift_right_logical(ux, sh + 8) == pref
                    byte = lax.shift_right_logical(ux, sh) & jnp.int32(255)
                    plsc.addupdate_scatter(
                        hist_v, [lane * 256 + byte], ones, mask=match
                    )
                    return 0

                lax.fori_loop(0, _NCH, scanr, 0)

            def merge(cb, carry):
                bcount, cum_before, total = carry

                def lsum(l, acc):
                    return acc + hist_v[pl.ds(l * 256 + cb * 16, 16)]

                mchunk = lax.fori_loop(0, 16, lsum, jnp.zeros((16,), jnp.int32))
                cum = plsc.cumsum(mchunk) + total
                sel = cum <= r_res
                bcount = bcount + jnp.sum(sel.astype(jnp.int32))
                cum_before = jnp.maximum(
                    cum_before, jnp.max(jnp.where(sel, cum, jnp.int32(0)))
                )
                total = total + jnp.sum(mchunk)
                return bcount, cum_before, total

            bidx, cum_before, _tot = lax.fori_loop(
                0, 16, merge, (jnp.int32(0), jnp.int32(0), jnp.int32(0))
            )
            pref = (pref * jnp.int32(256)) | bidx
            r_res = r_res - cum_before

        t_key = pref ^ _INT_MIN
        need = r_res

        def zpass(i, carry):
            key = key_v[pl.ds(i * 16, 16)]
            z = z_v[pl.ds(i * 16, 16)]
            ltm = key < t_key
            eqm = key == t_key
            m = eqm.astype(jnp.int32)
            c = plsc.cumsum(m)
            ord_excl = carry + (c - m)
            zero = jnp.logical_or(ltm, jnp.logical_and(eqm, ord_excl < need))
            z_v[pl.ds(i * 16, 16)] = jnp.where(zero, jnp.float32(0.0), z)
            return carry + jnp.sum(m)

        lax.fori_loop(0, _NCH, zpass, jnp.int32(0))

        pltpu.sync_copy(z_v, out_hbm)


_mask_kernel = pl.kernel(
    _body,
    out_type=jax.ShapeDtypeStruct((_N,), jnp.float32),
    mesh=plsc.VectorSubcoreMesh(core_axis_name="c", subcore_axis_name="s"),
    compiler_params=pltpu.CompilerParams(needs_layout_passes=False),
    scratch_types=[
        pltpu.VMEM((_N,), jnp.float32),
        pltpu.VMEM((_N,), jnp.float32),
        pltpu.VMEM((_N,), jnp.int32),
        pltpu.VMEM((4096,), jnp.int32),
    ],
)


def kernel(log_alpha):
    return _mask_kernel(log_alpha)


# speedup vs baseline: 3.5617x; 3.2707x over previous
import math

import jax
import jax.numpy as jnp
import numpy as np
from jax import lax
from jax.experimental import pallas as pl
from jax.experimental.pallas import tpu as pltpu
from jax.experimental.pallas import tpu_sc as plsc

_N = 32768
_T = 16
_E = _N // _T
_C = _E // 16
_U = 8
_BETA = 2.0 / 3.0
_MAGIC = 0.8
_X0 = (0.0 - (-0.1)) / (1.1 - (-0.1))
_LOGITS_BETA = (math.log(_X0) - math.log(1.0 - _X0)) * _BETA
_EPS = 1e-06
_INT_MIN = np.int32(-2147483648)
_M31 = np.int32(0x7FFFFFFF)


def _body(
    la_hbm,
    out_hbm,
    la_v,
    z_v,
    key_v,
    lhist_v,
    mhist_v,
    colrd_v,
    mrd_v,
    lsr_v,
    acc_v,
    tie_v,
    tierd_v,
    sh_lsum,
    sh_hist,
    sh_merged,
    sh_ties,
):
    w = lax.axis_index("s")
    lane = lax.iota(jnp.int32, 16)
    zeros16 = jnp.zeros((16,), jnp.int32)
    ones16 = jnp.ones((16,), jnp.int32)

    pltpu.sync_copy(la_hbm.at[pl.ds(w * _E, _E)], la_v)

    def zero_hist(j, _):
        for u in range(_U):
            lhist_v[pl.ds((j * _U + u) * 16, 16)] = zeros16
        return 0

    lax.fori_loop(0, 256 // _U, zero_hist, 0)

    def p1(i, acc):
        for u in range(_U):
            o = (i * _U + u) * 16
            x = la_v[pl.ds(o, 16)]
            at = jnp.clip(x - jnp.float32(_LOGITS_BETA), -15.0, 15.0)
            lv = jnp.clip(1.0 / (1.0 + jnp.exp(-at)), _EPS, 1.0 - _EPS)
            uu = x / jnp.float32(_BETA) * jnp.float32(_MAGIC)
            z_v[pl.ds(o, 16)] = 1.0 / (1.0 + jnp.exp(-uu))
            b = lax.bitcast_convert_type(x, jnp.int32)
            sgn = lax.shift_right_logical(b, 31)
            key = b ^ (sgn * _M31)
            key_v[pl.ds(o, 16)] = key
            ux = key ^ _INT_MIN
            byte = lax.shift_right_logical(ux, 24) & np.int32(255)
            plsc.addupdate_scatter(lhist_v, [lane * 256 + byte], ones16)
            acc = acc + lv
        return acc

    accv = lax.fori_loop(0, _C // _U, p1, jnp.zeros((16,), jnp.float32))
    acc_v[...] = accv
    pltpu.sync_copy(acc_v, sh_lsum.at[pl.ds(w * 16, 16)])

    def lane_merge_publish(rnd_sl):
        def lm(cb, _):
            def ls(l, a):
                return a + lhist_v[pl.ds(l * 256 + cb * 16, 16)]

            mhist_v[pl.ds(cb * 16, 16)] = lax.fori_loop(0, 16, ls, zeros16)
            return 0

        lax.fori_loop(0, 16, lm, 0)
        pltpu.sync_copy(mhist_v, sh_hist.at[pl.ds((rnd_sl * 16 + w) * 256, 256)])

    lane_merge_publish(0)
    plsc.subcore_barrier()

    pltpu.sync_copy(sh_lsum, lsr_v)

    def lsum(i, a):
        return a + lsr_v[pl.ds(i * 16, 16)]

    lc = jnp.sum(lax.fori_loop(0, 16, lsum, jnp.zeros((16,), jnp.float32)))
    t_i = lc.astype(jnp.int32)
    frac = lc - t_i.astype(jnp.float32)
    add1 = jnp.logical_or(
        frac > 0.5, jnp.logical_and(frac == 0.5, (t_i & 1) == 1)
    ).astype(jnp.int32)
    k = jnp.maximum(jnp.int32(1), t_i + add1)
    num_zeros = jnp.int32(_N) - k

    def merge_and_select(rnd, r_res):
        for t in range(16):
            pltpu.sync_copy(
                sh_hist.at[pl.ds((rnd * 16 + t) * 256 + w * 16, 16)],
                colrd_v.at[pl.ds(t * 16, 16)],
            )

        def cs(t, a):
            return a + colrd_v[pl.ds(t * 16, 16)]

        mg = lax.fori_loop(0, 16, cs, zeros16)
        mhist_v[pl.ds(0, 16)] = mg
        pltpu.sync_copy(
            mhist_v.at[pl.ds(0, 16)],
            sh_merged.at[pl.ds(rnd * 256 + w * 16, 16)],
        )
        plsc.subcore_barrier()
        pltpu.sync_copy(sh_merged.at[pl.ds(rnd * 256, 256)], mrd_v)

        def sel(cb, carry):
            bcount, cum_before, total = carry
            mchunk = mrd_v[pl.ds(cb * 16, 16)]
            cum = plsc.cumsum(mchunk) + total
            s = cum <= r_res
            bcount = bcount + jnp.sum(s.astype(jnp.int32))
            cum_before = jnp.maximum(
                cum_before, jnp.max(jnp.where(s, cum, jnp.int32(0)))
            )
            total = total + jnp.sum(mchunk)
            return bcount, cum_before, total

        bidx, cum_before, _tot = lax.fori_loop(
            0, 16, sel, (jnp.int32(0), jnp.int32(0), jnp.int32(0))
        )
        return bidx, r_res - cum_before

    bidx, r_res = merge_and_select(0, num_zeros)
    pref = bidx

    for rnd in range(1, 4):
        sh = 24 - 8 * rnd

        def zh(j, _):
            for u in range(_U):
                lhist_v[pl.ds((j * _U + u) * 16, 16)] = zeros16
            return 0

        lax.fori_loop(0, 256 // _U, zh, 0)

        def scanr(i, _, sh=sh, pref=pref):
            for u in range(_U):
                o = (i * _U + u) * 16
                key = key_v[pl.ds(o, 16)]
                ux = key ^ _INT_MIN
                match = lax.shift_right_logical(ux, sh + 8) == pref
                byte = lax.shift_right_logical(ux, sh) & np.int32(255)
                plsc.addupdate_scatter(
                    lhist_v, [lane * 256 + byte], ones16, mask=match
                )
            return 0

        lax.fori_loop(0, _C // _U, scanr, 0)
        lane_merge_publish(rnd)
        plsc.subcore_barrier()
        bidx, r_res = merge_and_select(rnd, r_res)
        pref = (pref * jnp.int32(256)) | bidx

    t_key = pref ^ _INT_MIN
    need = r_res

    lties = jnp.sum(plsc.load_gather(lhist_v, [lane * 256 + bidx]))
    tie_v[...] = jnp.where(lane == 0, lties, jnp.int32(0))
    pltpu.sync_copy(tie_v, sh_ties.at[pl.ds(w * 16, 16)])
    plsc.subcore_barrier()
    pltpu.sync_copy(sh_ties, tierd_v)

    def tb(t, a):
        row = tierd_v[pl.ds(t * 16, 16)]
        cnt = jnp.sum(jnp.where(lane == 0, row, jnp.int32(0)))
        return a + jnp.where(t < w, cnt, jnp.int32(0))

    ties_before = lax.fori_loop(0, 16, tb, jnp.int32(0))
    local_need = need - ties_before

    def zpass(i, carry):
        for u in range(_U):
            o = (i * _U + u) * 16
            key = key_v[pl.ds(o, 16)]
            z = z_v[pl.ds(o, 16)]
            ltm = key < t_key
            eqm = key == t_key
            m = eqm.astype(jnp.int32)
            c = plsc.cumsum(m)
            ord_excl = carry + (c - m)
            zero = jnp.logical_or(
                ltm, jnp.logical_and(eqm, ord_excl < local_need)
            )
            z_v[pl.ds(o, 16)] = jnp.where(zero, jnp.float32(0.0), z)
            carry = carry + jnp.sum(m)
        return carry

    lax.fori_loop(0, _C // _U, zpass, jnp.int32(0))

    pltpu.sync_copy(z_v, out_hbm.at[pl.ds(w * _E, _E)])


_mask_kernel = pl.kernel(
    _body,
    out_type=jax.ShapeDtypeStruct((_N,), jnp.float32),
    mesh=plsc.VectorSubcoreMesh(
        core_axis_name="c", subcore_axis_name="s", num_cores=1
    ),
    compiler_params=pltpu.CompilerParams(needs_layout_passes=False),
    scratch_types=[
        pltpu.VMEM((_E,), jnp.float32),
        pltpu.VMEM((_E,), jnp.float32),
        pltpu.VMEM((_E,), jnp.int32),
        pltpu.VMEM((4096,), jnp.int32),
        pltpu.VMEM((256,), jnp.int32),
        pltpu.VMEM((256,), jnp.int32),
        pltpu.VMEM((256,), jnp.int32),
        pltpu.VMEM((256,), jnp.float32),
        pltpu.VMEM((16,), jnp.float32),
        pltpu.VMEM((16,), jnp.int32),
        pltpu.VMEM((256,), jnp.int32),
        pltpu.VMEM_SHARED((256,), jnp.float32),
        pltpu.VMEM_SHARED((16384,), jnp.int32),
        pltpu.VMEM_SHARED((1024,), jnp.int32),
        pltpu.VMEM_SHARED((256,), jnp.int32),
    ],
)


def kernel(log_alpha):
    return _mask_kernel(log_alpha)


# speedup vs baseline: 4.6567x; 1.3075x over previous
import math

import jax
import jax.numpy as jnp
import numpy as np
from jax import lax
from jax.experimental import pallas as pl
from jax.experimental.pallas import tpu as pltpu
from jax.experimental.pallas import tpu_sc as plsc

_N = 32768
_T = 16
_E = _N // _T
_C = _E // 16
_U = 8
_BETA = 2.0 / 3.0
_MAGIC = 0.8
_X0 = (0.0 - (-0.1)) / (1.1 - (-0.1))
_LOGITS_BETA = (math.log(_X0) - math.log(1.0 - _X0)) * _BETA
_EPS = 1e-06
_INT_MIN = np.int32(-2147483648)
_M31 = np.int32(0x7FFFFFFF)


def _body(
    la_hbm,
    out_hbm,
    la_v,
    key_v,
    lhist_v,
    mhist_v,
    mrd_v,
    acc2_v,
    ls16_v,
    tie2_v,
    tie16_v,
    idx0_v,
    sh_m0,
    sh_m1,
    sh_m2,
    sh_m3,
    sh_ls,
    sh_tie,
):
    w = lax.axis_index("s")
    lane = lax.iota(jnp.int32, 16)
    zeros16 = jnp.zeros((16,), jnp.int32)
    zeros16f = jnp.zeros((16,), jnp.float32)
    ones16 = jnp.ones((16,), jnp.int32)
    sh_ms = [sh_m0, sh_m1, sh_m2, sh_m3]

    pltpu.sync_copy(la_hbm.at[pl.ds(w * _E, _E)], la_v)

    plsc.store_scatter(idx0_v, [zeros16], zeros16, mask=lane == 0)

    def zero_hist(j, _):
        for u in range(_U):
            lhist_v[pl.ds((j * _U + u) * 16, 16)] = zeros16
        return 0

    lax.fori_loop(0, 256 // _U, zero_hist, 0)

    for cb in range(16):
        mhist_v[0, pl.ds(cb * 16, 16)] = zeros16
    acc2_v[0, pl.ds(0, 16)] = zeros16f
    tie2_v[0, pl.ds(0, 16)] = zeros16

    for t in range(4):

        @pl.when(w == t)
        def _(t=t):
            pltpu.sync_copy(mhist_v, sh_ms[t])

    @pl.when(w == 4)
    def _():
        pltpu.sync_copy(acc2_v, sh_ls)

    @pl.when(w == 5)
    def _():
        pltpu.sync_copy(tie2_v, sh_tie)

    def p1(i, acc):
        for u in range(_U):
            o = (i * _U + u) * 16
            x = la_v[pl.ds(o, 16)]
            at = jnp.clip(x - jnp.float32(_LOGITS_BETA), -15.0, 15.0)
            lv = jnp.clip(1.0 / (1.0 + jnp.exp(-at)), _EPS, 1.0 - _EPS)
            b = lax.bitcast_convert_type(x, jnp.int32)
            sgn = lax.shift_right_logical(b, 31)
            key = b ^ (sgn * _M31)
            key_v[pl.ds(o, 16)] = key
            ux = key ^ _INT_MIN
            byte = lax.shift_right_logical(ux, 24) & np.int32(255)
            plsc.addupdate_scatter(lhist_v, [lane * 256 + byte], ones16)
            acc = acc + lv
        return acc

    accv = lax.fori_loop(0, _C // _U, p1, jnp.zeros((16,), jnp.float32))
    acc2_v[0, pl.ds(0, 16)] = accv

    plsc.subcore_barrier()

    def lane_merge(clear):
        def lm(cb, _):
            a = lhist_v[pl.ds(cb * 16, 16)]
            for l in range(1, 16):
                a = a + lhist_v[pl.ds(l * 256 + cb * 16, 16)]
            if clear:
                for l in range(16):
                    lhist_v[pl.ds(l * 256 + cb * 16, 16)] = zeros16
            mhist_v[0, pl.ds(cb * 16, 16)] = a
            return 0

        lax.fori_loop(0, 16, lm, 0)

    lane_merge(clear=True)
    pltpu.sync_copy(acc2_v, sh_ls.at[idx0_v], add=True)
    pltpu.sync_copy(mhist_v, sh_m0.at[idx0_v], add=True)
    plsc.subcore_barrier()

    pltpu.sync_copy(sh_ls.at[0], ls16_v)
    lc = jnp.sum(ls16_v[pl.ds(0, 16)])
    t_i = lc.astype(jnp.int32)
    frac = lc - t_i.astype(jnp.float32)
    add1 = jnp.logical_or(
        frac > 0.5, jnp.logical_and(frac == 0.5, (t_i & 1) == 1)
    ).astype(jnp.int32)
    k = jnp.maximum(jnp.int32(1), t_i + add1)
    num_zeros = jnp.int32(_N) - k

    def select(rnd, r_res):
        pltpu.sync_copy(sh_ms[rnd].at[0], mrd_v)

        def sel(cb, carry):
            bcount, cum_before, total = carry
            mchunk = mrd_v[pl.ds(cb * 16, 16)]
            cum = plsc.cumsum(mchunk) + total
            s = cum <= r_res
            bcount = bcount + jnp.sum(s.astype(jnp.int32))
            cum_before = jnp.maximum(
                cum_before, jnp.max(jnp.where(s, cum, jnp.int32(0)))
            )
            total = total + jnp.sum(mchunk)
            return bcount, cum_before, total

        bidx, cum_before, _tot = lax.fori_loop(
            0, 16, sel, (jnp.int32(0), jnp.int32(0), jnp.int32(0))
        )
        return bidx, r_res - cum_before

    bidx, r_res = select(0, num_zeros)
    pref = bidx

    for rnd in range(1, 4):
        sh = 24 - 8 * rnd

        def scanr(i, _, sh=sh, pref=pref):
            for u in range(_U):
                o = (i * _U + u) * 16
                key = key_v[pl.ds(o, 16)]
                ux = key ^ _INT_MIN
                match = lax.shift_right_logical(ux, sh + 8) == pref
                byte = lax.shift_right_logical(ux, sh) & np.int32(255)
                plsc.addupdate_scatter(
                    lhist_v, [lane * 256 + byte], ones16, mask=match
                )
            return 0

        lax.fori_loop(0, _C // _U, scanr, 0)
        lane_merge(clear=(rnd < 3))
        pltpu.sync_copy(mhist_v, sh_ms[rnd].at[idx0_v], add=True)
        plsc.subcore_barrier()
        bidx, r_res = select(rnd, r_res)
        pref = (pref * jnp.int32(256)) | bidx

    t_key = pref ^ _INT_MIN
    need = r_res

    lties = jnp.sum(plsc.load_gather(lhist_v, [lane * 256 + bidx]))
    tie2_v[0, pl.ds(0, 16)] = jnp.where(lane == w, lties, jnp.int32(0))
    pltpu.sync_copy(tie2_v, sh_tie.at[idx0_v], add=True)
    plsc.subcore_barrier()
    pltpu.sync_copy(sh_tie.at[0], tie16_v)
    tvec = tie16_v[pl.ds(0, 16)]
    ties_before = jnp.sum(jnp.where(lane < w, tvec, jnp.int32(0)))
    local_need = need - ties_before

    def zpass(i, carry):
        for u in range(_U):
            o = (i * _U + u) * 16
            x = la_v[pl.ds(o, 16)]
            key = key_v[pl.ds(o, 16)]
            uu = x / jnp.float32(_BETA) * jnp.float32(_MAGIC)
            z = 1.0 / (1.0 + jnp.exp(-uu))
            ltm = key < t_key
            eqm = key == t_key
            m = eqm.astype(jnp.int32)
            c = plsc.cumsum(m)
            ord_excl = carry + (c - m)
            zero = jnp.logical_or(
                ltm, jnp.logical_and(eqm, ord_excl < local_need)
            )
            la_v[pl.ds(o, 16)] = jnp.where(zero, jnp.float32(0.0), z)
            carry = carry + jnp.sum(m)
        return carry

    lax.fori_loop(0, _C // _U, zpass, jnp.int32(0))

    pltpu.sync_copy(la_v, out_hbm.at[pl.ds(w * _E, _E)])


_mask_kernel = pl.kernel(
    _body,
    out_type=jax.ShapeDtypeStruct((_N,), jnp.float32),
    mesh=plsc.VectorSubcoreMesh(
        core_axis_name="c", subcore_axis_name="s", num_cores=1
    ),
    compiler_params=pltpu.CompilerParams(needs_layout_passes=False),
    scratch_types=[
        pltpu.VMEM((_E,), jnp.float32),
        pltpu.VMEM((_E,), jnp.int32),
        pltpu.VMEM((4096,), jnp.int32),
        pltpu.VMEM((1, 256), jnp.int32),
        pltpu.VMEM((256,), jnp.int32),
        pltpu.VMEM((1, 16), jnp.float32),
        pltpu.VMEM((16,), jnp.float32),
        pltpu.VMEM((1, 16), jnp.int32),
        pltpu.VMEM((16,), jnp.int32),
        pltpu.VMEM((1,), jnp.int32),
        pltpu.VMEM_SHARED((1, 256), jnp.int32),
        pltpu.VMEM_SHARED((1, 256), jnp.int32),
        pltpu.VMEM_SHARED((1, 256), jnp.int32),
        pltpu.VMEM_SHARED((1, 256), jnp.int32),
        pltpu.VMEM_SHARED((1, 16), jnp.float32),
        pltpu.VMEM_SHARED((1, 16), jnp.int32),
    ],
)


def kernel(log_alpha):
    return _mask_kernel(log_alpha)
